# Initial kernel scaffold; baseline (speedup 1.0000x reference)
import jax
import jax.numpy as jnp
from jax.experimental import pallas as pl


def kernel(x_u, x_i, Wu, Wi, W1, b1, W2, b2, Wo, bo):
    raise NotImplementedError("write your pallas kernel here")


# speedup vs baseline: 2.6691x; 2.6691x over previous
import functools

import jax
import jax.numpy as jnp
from jax import lax
from jax.experimental import pallas as pl
from jax.experimental.pallas import tpu as pltpu
from jax.experimental.pallas import tpu_sc as plsc

_D = 128
_NC = 2
_NS = 16
_NW = _NC * _NS


def _sc_gather_pair(Wu, Wi, x_u, x_i):
    B = x_u.shape[0]
    b_per_w = B // _NW
    mesh = plsc.VectorSubcoreMesh(core_axis_name="c", subcore_axis_name="s")

    @functools.partial(
        pl.kernel,
        mesh=mesh,
        out_type=(
            jax.ShapeDtypeStruct((B, _D), jnp.float32),
            jax.ShapeDtypeStruct((B, _D), jnp.float32),
        ),
        scratch_types=[
            pltpu.VMEM((b_per_w,), jnp.int32),
            pltpu.VMEM((b_per_w, _D), jnp.float32),
            pltpu.SemaphoreType.DMA,
        ],
    )
    def gather_kernel(wu_hbm, wi_hbm, xu_hbm, xi_hbm, ou_hbm, oi_hbm,
                      idx_v, rows_v, sem):
        wid = lax.axis_index("s") * _NC + lax.axis_index("c")
        base = wid * b_per_w
        pltpu.sync_copy(xu_hbm.at[pl.ds(base, b_per_w)], idx_v)
        pltpu.async_copy(wu_hbm.at[idx_v], rows_v, sem).wait()
        pltpu.sync_copy(rows_v, ou_hbm.at[pl.ds(base, b_per_w)])
        pltpu.sync_copy(xi_hbm.at[pl.ds(base, b_per_w)], idx_v)
        pltpu.async_copy(wi_hbm.at[idx_v], rows_v, sem).wait()
        pltpu.sync_copy(rows_v, oi_hbm.at[pl.ds(base, b_per_w)])

    return gather_kernel(Wu, Wi, x_u, x_i)


def _mlp_body(hu_ref, hi_ref, w1_ref, b1_ref, w2_ref, b2_ref, wo_ref, bo_ref,
              o_ref):
    w1 = w1_ref[...]
    dn = (((1,), (1,)), ((), ()))
    a = lax.dot_general(hu_ref[...], w1[:, :_D], dn,
                        preferred_element_type=jnp.float32)
    a = a + lax.dot_general(hi_ref[...], w1[:, _D:], dn,
                            preferred_element_type=jnp.float32)
    a = a + b1_ref[...]
    a = jnp.where(a >= 0, a, 0.01 * a)
    b = lax.dot_general(a, w2_ref[...], dn,
                        preferred_element_type=jnp.float32)
    b = b + b2_ref[...]
    b = jnp.where(b >= 0, b, 0.01 * b)
    o_ref[...] = jnp.sum(b * wo_ref[...], axis=1, keepdims=True) + bo_ref[...]


def _mlp(hu, hi, W1, b1, W2, b2, Wo, bo, block_rows=1024):
    B = hu.shape[0]
    return pl.pallas_call(
        _mlp_body,
        grid=(B // block_rows,),
        in_specs=[
            pl.BlockSpec((block_rows, _D), lambda i: (i, 0)),
            pl.BlockSpec((block_rows, _D), lambda i: (i, 0)),
            pl.BlockSpec((2 * _D, 2 * _D), lambda i: (0, 0)),
            pl.BlockSpec((1, 2 * _D), lambda i: (0, 0)),
            pl.BlockSpec((2 * _D, 2 * _D), lambda i: (0, 0)),
            pl.BlockSpec((1, 2 * _D), lambda i: (0, 0)),
            pl.BlockSpec((1, 2 * _D), lambda i: (0, 0)),
            pl.BlockSpec((1, 1), lambda i: (0, 0)),
        ],
        out_specs=pl.BlockSpec((block_rows, 1), lambda i: (i, 0)),
        out_shape=jax.ShapeDtypeStruct((B, 1), jnp.float32),
    )(hu, hi, W1, b1.reshape(1, -1), W2, b2.reshape(1, -1), Wo,
      bo.reshape(1, 1))


def kernel(x_u, x_i, Wu, Wi, W1, b1, W2, b2, Wo, bo):
    hu, hi = _sc_gather_pair(Wu, Wi, x_u.astype(jnp.int32),
                             x_i.astype(jnp.int32))
    return _mlp(hu, hi, W1, b1, W2, b2, Wo, bo)
